# baseline (speedup 1.0000x reference)
import jax
import jax.numpy as jnp
from jax import lax
from jax.experimental import pallas as pl
from jax.experimental.pallas import tpu as pltpu
from jax.experimental.pallas import tpu_sc as plsc

B, N, K, D = 4, 2048, 8192, 3
NC, NS, L = 2, 16, 16
NW = NC * NS
PW = (B * N) // NW
G = 4
CHUNKS = K // L


def _sc_body(pts_hbm, keys_hbm, out_hbm, keys_v, pts_v, ids_v):
    wid = lax.axis_index("s") * NC + lax.axis_index("c")
    base = wid * PW
    BN = B * N
    pltpu.sync_copy(keys_hbm, keys_v)
    for coord in range(D):
        pltpu.sync_copy(pts_hbm.at[pl.ds(coord * BN + base, PW)],
                        pts_v.at[pl.ds(coord * PW, PW)])
    iota = lax.iota(jnp.int32, L)
    zeros = jnp.zeros((L,), jnp.int32)
    inf = jnp.full((L,), jnp.inf, jnp.float32)

    for pg in range(PW // (G * L)):
        px = [pts_v[pl.ds((pg * G + q) * L, L)] for q in range(G)]
        py = [pts_v[pl.ds(PW + (pg * G + q) * L, L)] for q in range(G)]
        pz = [pts_v[pl.ds(2 * PW + (pg * G + q) * L, L)] for q in range(G)]

        def chunk_fn(c, carry):
            best = list(carry[0])
            bidx = list(carry[1])
            off = c * L
            kx_v = keys_v[pl.ds(off, L)]
            ky_v = keys_v[pl.ds(K + off, L)]
            kz_v = keys_v[pl.ds(2 * K + off, L)]
            for j in range(L):
                kx = kx_v[j]
                ky = ky_v[j]
                kz = kz_v[j]
                kid = off + j
                for q in range(G):
                    dx = px[q] - kx
                    dy = py[q] - ky
                    dz = pz[q] - kz
                    dist = dx * dx + dy * dy + dz * dz
                    lt = dist < best[q]
                    best[q] = jnp.where(lt, dist, best[q])
                    bidx[q] = jnp.where(lt, kid, bidx[q])
            return tuple(best), tuple(bidx)

        best, bidx = lax.fori_loop(
            0, CHUNKS, chunk_fn,
            (tuple(inf for _ in range(G)), tuple(zeros for _ in range(G))))
        for q in range(G):
            ids_v[pl.ds((pg * G + q) * L, L)] = bidx[q]

    pltpu.sync_copy(ids_v, out_hbm.at[pl.ds(base, PW)])


def kernel(pointcloud, keys, table):
    del table
    pts_t = pointcloud.reshape(B * N, D).T.reshape(-1)
    keys_t = keys.T.reshape(-1)
    ids = pl.kernel(
        _sc_body,
        out_type=jax.ShapeDtypeStruct((B * N,), jnp.int32),
        mesh=plsc.VectorSubcoreMesh(core_axis_name="c", subcore_axis_name="s"),
        scratch_types=[
            pltpu.VMEM((D * K,), jnp.float32),
            pltpu.VMEM((D * PW,), jnp.float32),
            pltpu.VMEM((PW,), jnp.int32),
        ],
    )(pts_t, keys_t)
    return (ids.reshape(B, N, 1), pointcloud)


# speedup vs baseline: 1.1789x; 1.1789x over previous
import jax
import jax.numpy as jnp
from jax import lax
from jax.experimental import pallas as pl
from jax.experimental.pallas import tpu as pltpu
from jax.experimental.pallas import tpu_sc as plsc

B, N, K, D = 4, 2048, 8192, 3
BN = B * N
NC, NS, L = 2, 16, 16
NW = NC * NS
KW = K // NW
G = 4


def _sc_body(pts_hbm, krep_hbm, kid_hbm, best_hbm, bidx_hbm,
             pts_v, krep_v, kid_v, best_v, bidx_v):
    wid = lax.axis_index("s") * NC + lax.axis_index("c")
    kbase = wid * KW
    pltpu.sync_copy(pts_hbm, pts_v)
    for coord in range(D):
        pltpu.sync_copy(krep_hbm.at[pl.ds((coord * K + kbase) * L, KW * L)],
                        krep_v.at[pl.ds(coord * KW * L, KW * L)])
    pltpu.sync_copy(kid_hbm.at[pl.ds(kbase * L, KW * L)], kid_v)
    inf = jnp.full((L,), jnp.inf, jnp.float32)
    zeros = jnp.zeros((L,), jnp.int32)

    def pass_fn(pp, _):
        pbase = pp * (G * L)
        px = [pts_v[pl.ds(pbase + q * L, L)] for q in range(G)]
        py = [pts_v[pl.ds(BN + pbase + q * L, L)] for q in range(G)]
        pz = [pts_v[pl.ds(2 * BN + pbase + q * L, L)] for q in range(G)]

        def key_fn(kc, carry):
            best = list(carry[0])
            bidx = list(carry[1])
            tbase = kc * (L * L)
            for j in range(L):
                toff = tbase + j * L
                kx = krep_v[pl.ds(toff, L)]
                ky = krep_v[pl.ds(KW * L + toff, L)]
                kz = krep_v[pl.ds(2 * KW * L + toff, L)]
                kid = kid_v[pl.ds(toff, L)]
                for q in range(G):
                    dx = px[q] - kx
                    dy = py[q] - ky
                    dz = pz[q] - kz
                    dist = dx * dx + dy * dy + dz * dz
                    lt = dist < best[q]
                    best[q] = jnp.where(lt, dist, best[q])
                    bidx[q] = jnp.where(lt, kid, bidx[q])
            return tuple(best), tuple(bidx)

        best, bidx = lax.fori_loop(
            0, KW // L, key_fn,
            (tuple(inf for _ in range(G)), tuple(zeros for _ in range(G))))
        for q in range(G):
            best_v[pl.ds(pbase + q * L, L)] = best[q]
            bidx_v[pl.ds(pbase + q * L, L)] = bidx[q]
        return 0

    lax.fori_loop(0, BN // (G * L), pass_fn, 0)
    pltpu.sync_copy(best_v, best_hbm.at[wid])
    pltpu.sync_copy(bidx_v, bidx_hbm.at[wid])


def _merge_body(best_ref, bidx_ref, out_ref):
    run_best = best_ref[0]
    run_bidx = bidx_ref[0]
    for w in range(1, NW):
        b = best_ref[w]
        lt = b < run_best
        run_best = jnp.where(lt, b, run_best)
        run_bidx = jnp.where(lt, bidx_ref[w], run_bidx)
    out_ref[:, :] = run_bidx


def kernel(pointcloud, keys, table):
    del table
    pts_t = pointcloud.reshape(BN, D).T.reshape(-1)
    krep = jnp.broadcast_to(keys.T[:, :, None], (D, K, L)).reshape(-1)
    kid = jnp.broadcast_to(
        jnp.arange(K, dtype=jnp.int32)[:, None], (K, L)).reshape(-1)
    best_all, bidx_all = pl.kernel(
        _sc_body,
        out_type=(jax.ShapeDtypeStruct((NW, BN), jnp.float32),
                  jax.ShapeDtypeStruct((NW, BN), jnp.int32)),
        mesh=plsc.VectorSubcoreMesh(core_axis_name="c", subcore_axis_name="s"),
        scratch_types=[
            pltpu.VMEM((D * BN,), jnp.float32),
            pltpu.VMEM((D * KW * L,), jnp.float32),
            pltpu.VMEM((KW * L,), jnp.int32),
            pltpu.VMEM((BN,), jnp.float32),
            pltpu.VMEM((BN,), jnp.int32),
        ],
    )(pts_t, krep, kid)
    ids = pl.pallas_call(
        _merge_body,
        in_specs=[
            pl.BlockSpec((NW, 8, BN // 8), lambda: (0, 0, 0)),
            pl.BlockSpec((NW, 8, BN // 8), lambda: (0, 0, 0)),
        ],
        out_specs=pl.BlockSpec((8, BN // 8), lambda: (0, 0)),
        out_shape=jax.ShapeDtypeStruct((8, BN // 8), jnp.int32),
    )(best_all.reshape(NW, 8, BN // 8), bidx_all.reshape(NW, 8, BN // 8))
    return (ids.reshape(B, N, 1), pointcloud)


# speedup vs baseline: 5.0609x; 4.2931x over previous
import jax
import jax.numpy as jnp
from jax import lax
from jax.experimental import pallas as pl
from jax.experimental.pallas import tpu as pltpu
from jax.experimental.pallas import tpu_sc as plsc

B, N, K, D = 4, 2048, 8192, 3
BN = B * N
NC, NS, L = 2, 16, 16
NW = NC * NS
KW = K // NW
G = 4
KU = 4


def _sc_body(pts_hbm, krep_hbm, kid_hbm, best_hbm, bidx_hbm,
             pts_v, krep_v, kid_v, best_v, bidx_v):
    wid = lax.axis_index("s") * NC + lax.axis_index("c")
    kbase = wid * KW
    pltpu.sync_copy(pts_hbm, pts_v)
    for coord in range(D):
        pltpu.sync_copy(krep_hbm.at[pl.ds((coord * K + kbase) * L, KW * L)],
                        krep_v.at[pl.ds(coord * KW * L, KW * L)])
    pltpu.sync_copy(kid_hbm.at[pl.ds(kbase * L, KW * L)], kid_v)
    inf = jnp.full((L,), jnp.inf, jnp.float32)
    zeros = jnp.zeros((L,), jnp.int32)

    def pass_fn(pp, _):
        pbase = pp * (G * L)
        px = [pts_v[pl.ds(pbase + q * L, L)] for q in range(G)]
        py = [pts_v[pl.ds(BN + pbase + q * L, L)] for q in range(G)]
        pz = [pts_v[pl.ds(2 * BN + pbase + q * L, L)] for q in range(G)]

        def key_fn(kc, carry):
            best = list(carry[0])
            bidx = list(carry[1])
            tbase = kc * (KU * L)
            for j in range(KU):
                toff = tbase + j * L
                kx = krep_v[pl.ds(toff, L)]
                ky = krep_v[pl.ds(KW * L + toff, L)]
                kz = krep_v[pl.ds(2 * KW * L + toff, L)]
                kid = kid_v[pl.ds(toff, L)]
                for q in range(G):
                    dx = px[q] - kx
                    dy = py[q] - ky
                    dz = pz[q] - kz
                    dist = dx * dx + dy * dy + dz * dz
                    lt = dist < best[q]
                    best[q] = jnp.where(lt, dist, best[q])
                    bidx[q] = jnp.where(lt, kid, bidx[q])
            return tuple(best), tuple(bidx)

        best, bidx = lax.fori_loop(
            0, KW // KU, key_fn,
            (tuple(inf for _ in range(G)), tuple(zeros for _ in range(G))))
        for q in range(G):
            best_v[pl.ds(pbase + q * L, L)] = best[q]
            bidx_v[pl.ds(pbase + q * L, L)] = bidx[q]
        return 0

    lax.fori_loop(0, BN // (G * L), pass_fn, 0)
    pltpu.sync_copy(best_v, best_hbm.at[wid])
    pltpu.sync_copy(bidx_v, bidx_hbm.at[wid])


def _merge_body(best_ref, bidx_ref, out_ref):
    run_best = best_ref[0]
    run_bidx = bidx_ref[0]
    for w in range(1, NW):
        b = best_ref[w]
        lt = b < run_best
        run_best = jnp.where(lt, b, run_best)
        run_bidx = jnp.where(lt, bidx_ref[w], run_bidx)
    out_ref[:, :] = run_bidx


def kernel(pointcloud, keys, table):
    del table
    pts_t = pointcloud.reshape(BN, D).T.reshape(-1)
    krep = jnp.broadcast_to(keys.T[:, :, None], (D, K, L)).reshape(-1)
    kid = jnp.broadcast_to(
        jnp.arange(K, dtype=jnp.int32)[:, None], (K, L)).reshape(-1)
    best_all, bidx_all = pl.kernel(
        _sc_body,
        out_type=(jax.ShapeDtypeStruct((NW, BN), jnp.float32),
                  jax.ShapeDtypeStruct((NW, BN), jnp.int32)),
        mesh=plsc.VectorSubcoreMesh(core_axis_name="c", subcore_axis_name="s"),
        scratch_types=[
            pltpu.VMEM((D * BN,), jnp.float32),
            pltpu.VMEM((D * KW * L,), jnp.float32),
            pltpu.VMEM((KW * L,), jnp.int32),
            pltpu.VMEM((BN,), jnp.float32),
            pltpu.VMEM((BN,), jnp.int32),
        ],
    )(pts_t, krep, kid)
    ids = pl.pallas_call(
        _merge_body,
        in_specs=[
            pl.BlockSpec((NW, 8, BN // 8), lambda: (0, 0, 0)),
            pl.BlockSpec((NW, 8, BN // 8), lambda: (0, 0, 0)),
        ],
        out_specs=pl.BlockSpec((8, BN // 8), lambda: (0, 0)),
        out_shape=jax.ShapeDtypeStruct((8, BN // 8), jnp.int32),
    )(best_all.reshape(NW, 8, BN // 8), bidx_all.reshape(NW, 8, BN // 8))
    return (ids.reshape(B, N, 1), pointcloud)


# speedup vs baseline: 10.4967x; 2.0741x over previous
import jax
import jax.numpy as jnp
from jax import lax
from jax.experimental import pallas as pl
from jax.experimental.pallas import tpu as pltpu
from jax.experimental.pallas import tpu_sc as plsc

B, N, K, D = 4, 2048, 8192, 3
BN = B * N
NC, NS, L = 2, 16, 16
NW = NC * NS
KW = K // NW
G = 4
KU = 4
SP = 3072
TCP = BN - SP
PN = 256


def _sc_body(pts_hbm, krep_hbm, kid_hbm, best_hbm, bidx_hbm,
             pts_v, krep_v, kid_v, best_v, bidx_v):
    wid = lax.axis_index("s") * NC + lax.axis_index("c")
    kbase = wid * KW
    pltpu.sync_copy(pts_hbm, pts_v)
    for coord in range(D):
        pltpu.sync_copy(krep_hbm.at[pl.ds((coord * K + kbase) * L, KW * L)],
                        krep_v.at[pl.ds(coord * KW * L, KW * L)])
    pltpu.sync_copy(kid_hbm.at[pl.ds(kbase * L, KW * L)], kid_v)
    inf = jnp.full((L,), jnp.inf, jnp.float32)
    zeros = jnp.zeros((L,), jnp.int32)

    def pass_fn(pp, _):
        pbase = pp * (G * L)
        px = [pts_v[pl.ds(pbase + q * L, L)] for q in range(G)]
        py = [pts_v[pl.ds(SP + pbase + q * L, L)] for q in range(G)]
        pz = [pts_v[pl.ds(2 * SP + pbase + q * L, L)] for q in range(G)]

        def key_fn(kc, carry):
            best = list(carry[0])
            bidx = list(carry[1])
            tbase = kc * (KU * L)
            for j in range(KU):
                toff = tbase + j * L
                kx = krep_v[pl.ds(toff, L)]
                ky = krep_v[pl.ds(KW * L + toff, L)]
                kz = krep_v[pl.ds(2 * KW * L + toff, L)]
                kid = kid_v[pl.ds(toff, L)]
                for q in range(G):
                    dx = px[q] - kx
                    dy = py[q] - ky
                    dz = pz[q] - kz
                    dist = dx * dx + dy * dy + dz * dz
                    lt = dist < best[q]
                    best[q] = jnp.where(lt, dist, best[q])
                    bidx[q] = jnp.where(lt, kid, bidx[q])
            return tuple(best), tuple(bidx)

        best, bidx = lax.fori_loop(
            0, KW // KU, key_fn,
            (tuple(inf for _ in range(G)), tuple(zeros for _ in range(G))))
        for q in range(G):
            best_v[pl.ds(pbase + q * L, L)] = best[q]
            bidx_v[pl.ds(pbase + q * L, L)] = bidx[q]
        return 0

    lax.fori_loop(0, SP // (G * L), pass_fn, 0)
    pltpu.sync_copy(best_v, best_hbm.at[wid])
    pltpu.sync_copy(bidx_v, bidx_hbm.at[wid])


def _merge_body(best_ref, bidx_ref, out_ref):
    run_best = best_ref[0]
    run_bidx = bidx_ref[0]
    for w in range(1, NW):
        b = best_ref[w]
        lt = b < run_best
        run_best = jnp.where(lt, b, run_best)
        run_bidx = jnp.where(lt, bidx_ref[w], run_bidx)
    out_ref[:, :] = run_bidx


def _tc_body(pts_ref, keys_t_ref, out_ref):
    px = pts_ref[:, 0:1]
    py = pts_ref[:, 1:2]
    pz = pts_ref[:, 2:3]
    kx = keys_t_ref[0:1, :]
    ky = keys_t_ref[1:2, :]
    kz = keys_t_ref[2:3, :]
    dx = px - kx
    dy = py - ky
    dz = pz - kz
    dist = dx * dx + dy * dy + dz * dz
    out_ref[:, :] = jnp.argmin(dist, axis=1, keepdims=True).astype(jnp.int32)


def kernel(pointcloud, keys, table):
    del table
    pts = pointcloud.reshape(BN, D)
    pts_sc = pts[TCP:].T.reshape(-1)
    krep = jnp.broadcast_to(keys.T[:, :, None], (D, K, L)).reshape(-1)
    kid = jnp.broadcast_to(
        jnp.arange(K, dtype=jnp.int32)[:, None], (K, L)).reshape(-1)
    best_all, bidx_all = pl.kernel(
        _sc_body,
        out_type=(jax.ShapeDtypeStruct((NW, SP), jnp.float32),
                  jax.ShapeDtypeStruct((NW, SP), jnp.int32)),
        mesh=plsc.VectorSubcoreMesh(core_axis_name="c", subcore_axis_name="s"),
        scratch_types=[
            pltpu.VMEM((D * SP,), jnp.float32),
            pltpu.VMEM((D * KW * L,), jnp.float32),
            pltpu.VMEM((KW * L,), jnp.int32),
            pltpu.VMEM((SP,), jnp.float32),
            pltpu.VMEM((SP,), jnp.int32),
        ],
    )(pts_sc, krep, kid)

    ids_tc = pl.pallas_call(
        _tc_body,
        grid=(TCP // PN,),
        in_specs=[
            pl.BlockSpec((PN, D), lambda i: (i, 0)),
            pl.BlockSpec((D, K), lambda i: (0, 0)),
        ],
        out_specs=pl.BlockSpec((PN, 1), lambda i: (i, 0)),
        out_shape=jax.ShapeDtypeStruct((TCP, 1), jnp.int32),
    )(pts[:TCP], keys.T)

    ids_sc = pl.pallas_call(
        _merge_body,
        in_specs=[
            pl.BlockSpec((NW, 8, SP // 8), lambda: (0, 0, 0)),
            pl.BlockSpec((NW, 8, SP // 8), lambda: (0, 0, 0)),
        ],
        out_specs=pl.BlockSpec((8, SP // 8), lambda: (0, 0)),
        out_shape=jax.ShapeDtypeStruct((8, SP // 8), jnp.int32),
    )(best_all.reshape(NW, 8, SP // 8), bidx_all.reshape(NW, 8, SP // 8))

    ids = jnp.concatenate([ids_tc.reshape(-1), ids_sc.reshape(-1)])
    return (ids.reshape(B, N, 1), pointcloud)


# speedup vs baseline: 11.4386x; 1.0897x over previous
import jax
import jax.numpy as jnp
from jax import lax
from jax.experimental import pallas as pl
from jax.experimental.pallas import tpu as pltpu
from jax.experimental.pallas import tpu_sc as plsc

B, N, K, D = 4, 2048, 8192, 3
BN = B * N
NC, NS, L = 2, 16, 16
NW = NC * NS
G = 4
KU = 4
CHUNKS = K // L
SP = 3072
PWC = SP // NW
TCP = BN - SP
PN = 256


def _sc_body(pts_hbm, keys_hbm, out_hbm, pts_v, keys_v, ids_v, rot_v, rot_i):
    wid = lax.axis_index("s") * NC + lax.axis_index("c")
    pltpu.sync_copy(keys_hbm, keys_v)
    pltpu.sync_copy(pts_hbm.at[pl.ds((TCP + wid * PWC) * D, PWC * D)], pts_v)
    iota = lax.iota(jnp.int32, L)
    inf = jnp.full((L,), jnp.inf, jnp.float32)
    zeros = jnp.zeros((L,), jnp.int32)

    def group_fn(g, _):
        res = zeros
        vs = [pts_v[pl.ds(g * (D * L) + v * L, L)] for v in range(D)]
        for sb in range(L // G):
            px, py, pz = [], [], []
            for q in range(G):
                j = D * (sb * G + q)
                px.append(jnp.full((L,), vs[j // L][j % L], jnp.float32))
                py.append(jnp.full((L,), vs[(j + 1) // L][(j + 1) % L],
                                   jnp.float32))
                pz.append(jnp.full((L,), vs[(j + 2) // L][(j + 2) % L],
                                   jnp.float32))

            def chunk_fn(c, carry):
                best = list(carry[0])
                bidx = list(carry[1])
                for u in range(KU):
                    off = (c * KU + u) * L
                    kx = keys_v[pl.ds(off, L)]
                    ky = keys_v[pl.ds(K + off, L)]
                    kz = keys_v[pl.ds(2 * K + off, L)]
                    kidx = iota + off
                    for q in range(G):
                        dx = px[q] - kx
                        dy = py[q] - ky
                        dz = pz[q] - kz
                        dist = dx * dx + dy * dy + dz * dz
                        lt = dist < best[q]
                        best[q] = jnp.where(lt, dist, best[q])
                        bidx[q] = jnp.where(lt, kidx, bidx[q])
                return tuple(best), tuple(bidx)

            best, bidx = lax.fori_loop(
                0, CHUNKS // KU, chunk_fn,
                (tuple(inf for _ in range(G)), tuple(zeros for _ in range(G))))

            for q in range(G):
                b, bi = best[q], bidx[q]
                for r in (8, 4, 2, 1):
                    rot_v[pl.ds(0, L)] = b
                    rot_v[pl.ds(L, L)] = b
                    rot_i[pl.ds(0, L)] = bi
                    rot_i[pl.ds(L, L)] = bi
                    rb = rot_v[pl.ds(r, L)]
                    ri = rot_i[pl.ds(r, L)]
                    take = (rb < b) | ((rb == b) & (ri < bi))
                    b = jnp.where(take, rb, b)
                    bi = jnp.where(take, ri, bi)
                res = jnp.where(iota == sb * G + q, bi, res)
        ids_v[pl.ds(g * L, L)] = res
        return 0

    lax.fori_loop(0, PWC // L, group_fn, 0)
    pltpu.sync_copy(ids_v, out_hbm.at[pl.ds(wid * PWC, PWC)])


def _tc_body(pts_ref, keys_t_ref, out_ref):
    px = pts_ref[:, 0:1]
    py = pts_ref[:, 1:2]
    pz = pts_ref[:, 2:3]
    kx = keys_t_ref[0:1, :]
    ky = keys_t_ref[1:2, :]
    kz = keys_t_ref[2:3, :]
    dx = px - kx
    dy = py - ky
    dz = pz - kz
    dist = dx * dx + dy * dy + dz * dz
    out_ref[:, :] = jnp.argmin(dist, axis=1, keepdims=True).astype(jnp.int32)


def kernel(pointcloud, keys, table):
    del table
    pts_flat = pointcloud.reshape(-1)
    keys_t = keys.T

    ids_sc = pl.kernel(
        _sc_body,
        out_type=jax.ShapeDtypeStruct((SP,), jnp.int32),
        mesh=plsc.VectorSubcoreMesh(core_axis_name="c", subcore_axis_name="s"),
        scratch_types=[
            pltpu.VMEM((PWC * D,), jnp.float32),
            pltpu.VMEM((D * K,), jnp.float32),
            pltpu.VMEM((PWC,), jnp.int32),
            pltpu.VMEM((2 * L,), jnp.float32),
            pltpu.VMEM((2 * L,), jnp.int32),
        ],
    )(pts_flat, keys_t.reshape(-1))

    ids_tc = pl.pallas_call(
        _tc_body,
        grid=(TCP // PN,),
        in_specs=[
            pl.BlockSpec((PN, D), lambda i: (i, 0)),
            pl.BlockSpec((D, K), lambda i: (0, 0)),
        ],
        out_specs=pl.BlockSpec((PN, 1), lambda i: (i, 0)),
        out_shape=jax.ShapeDtypeStruct((TCP, 1), jnp.int32),
    )(pointcloud.reshape(BN, D)[:TCP], keys_t)

    ids = jnp.concatenate([ids_tc.reshape(-1), ids_sc])
    return (ids.reshape(B, N, 1), pointcloud)


# speedup vs baseline: 12.3975x; 1.0838x over previous
import jax
import jax.numpy as jnp
from jax import lax
from jax.experimental import pallas as pl
from jax.experimental.pallas import tpu as pltpu
from jax.experimental.pallas import tpu_sc as plsc

B, N, K, D = 4, 2048, 8192, 3
BN = B * N
NC, NS, L = 2, 16, 16
NW = NC * NS
G = 4
KU = 4
CHUNKS = K // L
SP = 2560
PWC = SP // NW
TCP = BN - SP
PN = 256


def _sc_body(pts_hbm, keys_hbm, out_hbm, pts_v, keys_v, ids_v, rot_v, rot_i):
    wid = lax.axis_index("s") * NC + lax.axis_index("c")
    pltpu.sync_copy(keys_hbm, keys_v)
    pltpu.sync_copy(pts_hbm.at[pl.ds((TCP + wid * PWC) * D, PWC * D)], pts_v)
    iota = lax.iota(jnp.int32, L)
    inf = jnp.full((L,), jnp.inf, jnp.float32)
    zeros = jnp.zeros((L,), jnp.int32)

    def group_fn(g, _):
        res = zeros
        vs = [pts_v[pl.ds(g * (D * L) + v * L, L)] for v in range(D)]
        for sb in range(L // G):
            px, py, pz = [], [], []
            for q in range(G):
                j = D * (sb * G + q)
                px.append(jnp.full((L,), vs[j // L][j % L], jnp.float32))
                py.append(jnp.full((L,), vs[(j + 1) // L][(j + 1) % L],
                                   jnp.float32))
                pz.append(jnp.full((L,), vs[(j + 2) // L][(j + 2) % L],
                                   jnp.float32))

            def chunk_fn(c, carry):
                best = list(carry[0])
                bidx = list(carry[1])
                for u in range(KU):
                    off = (c * KU + u) * L
                    kx = keys_v[pl.ds(off, L)]
                    ky = keys_v[pl.ds(K + off, L)]
                    kz = keys_v[pl.ds(2 * K + off, L)]
                    kidx = iota + off
                    for q in range(G):
                        dx = px[q] - kx
                        dy = py[q] - ky
                        dz = pz[q] - kz
                        dist = dx * dx + dy * dy + dz * dz
                        lt = dist < best[q]
                        best[q] = jnp.where(lt, dist, best[q])
                        bidx[q] = jnp.where(lt, kidx, bidx[q])
                return tuple(best), tuple(bidx)

            best, bidx = lax.fori_loop(
                0, CHUNKS // KU, chunk_fn,
                (tuple(inf for _ in range(G)), tuple(zeros for _ in range(G))))

            for q in range(G):
                b, bi = best[q], bidx[q]
                for r in (8, 4, 2, 1):
                    rot_v[pl.ds(0, L)] = b
                    rot_v[pl.ds(L, L)] = b
                    rot_i[pl.ds(0, L)] = bi
                    rot_i[pl.ds(L, L)] = bi
                    rb = rot_v[pl.ds(r, L)]
                    ri = rot_i[pl.ds(r, L)]
                    take = (rb < b) | ((rb == b) & (ri < bi))
                    b = jnp.where(take, rb, b)
                    bi = jnp.where(take, ri, bi)
                res = jnp.where(iota == sb * G + q, bi, res)
        ids_v[pl.ds(g * L, L)] = res
        return 0

    lax.fori_loop(0, PWC // L, group_fn, 0)
    pltpu.sync_copy(ids_v, out_hbm.at[pl.ds(wid * PWC, PWC)])


def _tc_body(pts_ref, keys_t_ref, out_ref):
    px = pts_ref[:, 0:1]
    py = pts_ref[:, 1:2]
    pz = pts_ref[:, 2:3]
    kx = keys_t_ref[0:1, :]
    ky = keys_t_ref[1:2, :]
    kz = keys_t_ref[2:3, :]
    dx = px - kx
    dy = py - ky
    dz = pz - kz
    dist = dx * dx + dy * dy + dz * dz
    out_ref[:, :] = jnp.argmin(dist, axis=1, keepdims=True).astype(jnp.int32)


def kernel(pointcloud, keys, table):
    del table
    pts_flat = pointcloud.reshape(-1)
    keys_flat = jnp.concatenate([keys[:, 0], keys[:, 1], keys[:, 2]])
    keys_t = keys_flat.reshape(D, K)

    ids_sc = pl.kernel(
        _sc_body,
        out_type=jax.ShapeDtypeStruct((SP,), jnp.int32),
        mesh=plsc.VectorSubcoreMesh(core_axis_name="c", subcore_axis_name="s"),
        scratch_types=[
            pltpu.VMEM((PWC * D,), jnp.float32),
            pltpu.VMEM((D * K,), jnp.float32),
            pltpu.VMEM((PWC,), jnp.int32),
            pltpu.VMEM((2 * L,), jnp.float32),
            pltpu.VMEM((2 * L,), jnp.int32),
        ],
    )(pts_flat, keys_flat)

    ids_tc = pl.pallas_call(
        _tc_body,
        grid=(TCP // PN,),
        in_specs=[
            pl.BlockSpec((PN, D), lambda i: (i, 0)),
            pl.BlockSpec((D, K), lambda i: (0, 0)),
        ],
        out_specs=pl.BlockSpec((PN, 1), lambda i: (i, 0)),
        out_shape=jax.ShapeDtypeStruct((TCP, 1), jnp.int32),
    )(pointcloud.reshape(BN, D)[:TCP], keys_t)

    ids = jnp.concatenate([ids_tc.reshape(-1), ids_sc])
    return (ids.reshape(B, N, 1), pointcloud)
